# baseline (speedup 1.0000x reference)
import functools

import jax
import jax.numpy as jnp
from jax import lax
from jax.experimental import pallas as pl
from jax.experimental.pallas import tpu as pltpu
from jax.experimental.pallas import tpu_sc as plsc

_NUM_CORES = 2
_NUM_SUBCORES = 16
_NUM_WORKERS = _NUM_CORES * _NUM_SUBCORES
_CW = 512
_FLUSH = 64
_GROUP = 16
_SEG_PAD = _FLUSH
_BND = 96


def _chunk_split(V):
    n_chunks = (V + _CW - 1) // _CW
    k_lo = [w * n_chunks // _NUM_WORKERS for w in range(_NUM_WORKERS + 1)]
    return n_chunks, k_lo


@functools.lru_cache(maxsize=None)
def _make_sweep(V, D, B):
    si_cap = B + 2048 + 16
    mesh = plsc.VectorSubcoreMesh(
        core_axis_name="c",
        subcore_axis_name="s",
        num_cores=_NUM_CORES,
        num_subcores=_NUM_SUBCORES,
    )

    @functools.partial(
        pl.kernel,
        mesh=mesh,
        compiler_params=pltpu.CompilerParams(needs_layout_passes=False),
        out_type=jax.ShapeDtypeStruct(((B + _NUM_WORKERS * _SEG_PAD) * D,), jnp.float32),
        scratch_types=[
            pltpu.VMEM((D, _CW), jnp.float32),
            pltpu.VMEM((D, V - (V // _CW) * _CW or _CW), jnp.float32),
            pltpu.VMEM((si_cap,), jnp.int32),
            pltpu.VMEM((_BND,), jnp.int32),
            pltpu.VMEM((_FLUSH * D,), jnp.float32),
        ],
    )
    def sweep_kernel(
        si_hbm, bounds_hbm, tt_hbm, tail_hbm, rows_hbm,
        chunk_v, tail_v, si_v, bnd_v, stage_v,
    ):
        wid = lax.axis_index("s") * _NUM_CORES + lax.axis_index("c")
        pltpu.sync_copy(bounds_hbm, bnd_v)
        bv = bnd_v[pl.ds(wid, 16)]
        lo = bv[0]
        hi = bv[1]
        kv = bnd_v[pl.ds(wid + 33, 16)]
        k_lo = kv[0]
        k_hi = kv[1]
        lo8 = (lo // 8) * 8
        n_blk = (hi - lo8 + 1023) // 1024

        def load_blk(t, _):
            pltpu.sync_copy(
                si_hbm.at[pl.ds(lo8 + t * 1024, 1024)],
                si_v.at[pl.ds(t * 1024, 1024)],
            )
            return 0

        lax.fori_loop(0, n_blk, load_blk, 0)

        f16 = lax.iota(jnp.int32, 16)
        write_base = lo + wid * _SEG_PAD

        n_full = V // _CW
        pltpu.sync_copy(tail_hbm, tail_v)

        def make_entry_loop(src_v, start, end):
            def ent_cond(c):
                j1, q1, fl1 = c
                pv = si_v[pl.ds(j1 - lo8, 16)]
                return (j1 < hi) & (pv[0] < end)

            def ent_body(c):
                j1, q1, fl1 = c
                pv = si_v[pl.ds(j1 - lo8, 16)]
                col = pv[0] - start
                for u in range(D // 16):
                    vals = plsc.load_gather(
                        src_v, [f16 + u * 16, jnp.broadcast_to(col, (16,))]
                    )
                    stage_v[pl.ds(q1 * D + u * 16, 16)] = vals
                q2 = q1 + 1

                @pl.when(q2 == _FLUSH)
                def _flush():
                    pltpu.sync_copy(
                        stage_v,
                        rows_hbm.at[
                            pl.ds((write_base + fl1 * _FLUSH) * D, _FLUSH * D)
                        ],
                    )

                fl2 = jnp.where(q2 == _FLUSH, fl1 + 1, fl1)
                return j1 + 1, jnp.where(q2 == _FLUSH, 0, q2), fl2

            return ent_cond, ent_body

        def chunk_body(k, carry):
            start = pl.multiple_of(k * _CW, 128)
            pltpu.sync_copy(tt_hbm.at[:, pl.ds(start, _CW)], chunk_v)
            cond, body = make_entry_loop(chunk_v, k * _CW, (k + 1) * _CW)
            return lax.while_loop(cond, body, carry)

        init = (lo, jnp.int32(0), jnp.int32(0))
        carry = lax.fori_loop(k_lo, jnp.minimum(k_hi, n_full), chunk_body, init)
        t_cond, t_body = make_entry_loop(tail_v, n_full * _CW, V)
        j, q, fl = lax.while_loop(t_cond, t_body, carry)

        @pl.when(q > 0)
        def _tail():
            pltpu.sync_copy(
                stage_v,
                rows_hbm.at[pl.ds((write_base + fl * _FLUSH) * D, _FLUSH * D)],
            )

    return sweep_kernel


@functools.lru_cache(maxsize=None)
def _make_permute(V, D, B, rows_len):
    b_per_w = B // _NUM_WORKERS
    n_groups = b_per_w // _GROUP
    mesh = plsc.VectorSubcoreMesh(
        core_axis_name="c",
        subcore_axis_name="s",
        num_cores=_NUM_CORES,
        num_subcores=_NUM_SUBCORES,
    )

    @functools.partial(
        pl.kernel,
        mesh=mesh,
        out_type=jax.ShapeDtypeStruct((B, D), jnp.float32),
        scratch_types=[
            pltpu.VMEM((b_per_w,), jnp.int32),
            pltpu.VMEM((b_per_w, D), jnp.float32),
            pltpu.SemaphoreType.DMA,
        ],
    )
    def permute_kernel(rank_hbm, rows_hbm, out_hbm, rank_v, rows_v, sem):
        wid = lax.axis_index("s") * _NUM_CORES + lax.axis_index("c")
        base = wid * b_per_w
        pltpu.sync_copy(rank_hbm.at[pl.ds(base, b_per_w)], rank_v)

        def step(g, _):
            rv = rank_v[pl.ds(g * _GROUP, _GROUP)]
            copies = []
            for u in range(_GROUP):
                copies.append(
                    pltpu.async_copy(
                        rows_hbm.at[pl.ds(rv[u] * D, D)],
                        rows_v.at[g * _GROUP + u],
                        sem,
                    )
                )
            for c in copies:
                c.wait()
            return 0

        lax.fori_loop(0, n_groups, step, 0)
        pltpu.sync_copy(rows_v, out_hbm.at[pl.ds(base, b_per_w)])

    return permute_kernel


def kernel(provider_ids, table):
    (B,) = provider_ids.shape
    V, D = table.shape
    idx = provider_ids.astype(jnp.int32)

    si = jnp.sort(idx)
    order = jnp.argsort(idx)
    rank = jnp.zeros((B,), jnp.int32).at[order].set(jnp.arange(B, dtype=jnp.int32))

    n_chunks, k_lo = _chunk_split(V)
    k_lo_arr = jnp.asarray(k_lo, jnp.int32)
    seg_starts = jnp.minimum(k_lo_arr[:_NUM_WORKERS] * _CW, V)
    bounds = jnp.searchsorted(si, seg_starts, side="left").astype(jnp.int32)
    bounds = jnp.concatenate([bounds, jnp.array([B], jnp.int32)])
    seg_of = (
        jnp.searchsorted(bounds[1 : _NUM_WORKERS + 1], rank, side="right")
        .astype(jnp.int32)
    )
    rank_padded = rank + _SEG_PAD * seg_of

    bnd = jnp.concatenate(
        [bounds, k_lo_arr, jnp.zeros((_BND - 2 * (_NUM_WORKERS + 1),), jnp.int32)]
    )
    si_padded = jnp.concatenate([si, jnp.full((2048 + 16,), V, jnp.int32)])

    tail = table[(V // _CW) * _CW :, :].T
    rows_flat = _make_sweep(V, D, B)(si_padded, bnd, table.T, tail)
    return _make_permute(V, D, B, rows_flat.shape[0])(rank_padded, rows_flat)


# speedup vs baseline: 1.4513x; 1.4513x over previous
import functools

import jax
import jax.numpy as jnp
from jax import lax
from jax.experimental import pallas as pl
from jax.experimental.pallas import tpu as pltpu
from jax.experimental.pallas import tpu_sc as plsc

_NUM_CORES = 2
_NUM_SUBCORES = 16
_NUM_WORKERS = _NUM_CORES * _NUM_SUBCORES
_GROUP = 16


@functools.lru_cache(maxsize=None)
def _make_kernel(T, D, B):
    b_per_w = B // _NUM_WORKERS
    n_groups = b_per_w // _GROUP
    mesh = plsc.VectorSubcoreMesh(
        core_axis_name="c",
        subcore_axis_name="s",
        num_cores=_NUM_CORES,
        num_subcores=_NUM_SUBCORES,
    )

    @functools.partial(
        pl.kernel,
        mesh=mesh,
        out_type=jax.ShapeDtypeStruct((B // 8, 8, D), jnp.float32),
        scratch_types=[
            pltpu.VMEM((b_per_w,), jnp.int32),
            pltpu.VMEM((b_per_w,), jnp.int32),
            pltpu.VMEM((b_per_w // 8, 8, D), jnp.float32),
            pltpu.SemaphoreType.DMA,
        ],
    )
    def gather_kernel(t_hbm, s_hbm, table_hbm, out_hbm, t_sm, s_sm, rows_v, sem):
        wid = lax.axis_index("s") * _NUM_CORES + lax.axis_index("c")
        base = wid * b_per_w
        pltpu.sync_copy(t_hbm.at[pl.ds(base, b_per_w)], t_sm)
        pltpu.sync_copy(s_hbm.at[pl.ds(base, b_per_w)], s_sm)

        def step(g, _):
            tv = t_sm[pl.ds(g * _GROUP, _GROUP)]
            sv = s_sm[pl.ds(g * _GROUP, _GROUP)]
            copies = []
            for u in range(_GROUP):
                b = g * _GROUP + u
                copies.append(
                    pltpu.async_copy(
                        table_hbm.at[tv[u], sv[u]],
                        rows_v.at[b // 8, b % 8],
                        sem,
                    )
                )
            for c in copies:
                c.wait()
            return 0

        lax.fori_loop(0, n_groups, step, 0)
        pltpu.sync_copy(rows_v, out_hbm.at[pl.ds(base // 8, b_per_w // 8)])

    return gather_kernel


def kernel(provider_ids, table):
    (B,) = provider_ids.shape
    V, D = table.shape
    idx = provider_ids.astype(jnp.int32)
    t = idx // 8
    s = idx - t * 8
    table3 = table.reshape(V // 8, 8, D)
    out3 = _make_kernel(V // 8, D, B)(t, s, table3)
    return out3.reshape(B, D)
